# baseline (speedup 1.0000x reference)
import functools

import jax
import jax.numpy as jnp
from jax import lax
from jax.experimental import pallas as pl
from jax.experimental.pallas import tpu as pltpu
from jax.experimental.pallas import tpu_sc as plsc

VOCAB = 1000000
MAX_LEN = 200
EMBED_DIM = 64
BATCH = 1024
SEQ_LEN = 200

N = BATCH * SEQ_LEN
WIDE = 2 * EMBED_DIM
_INFO = plsc.get_sparse_core_info()
NC = _INFO.num_cores
NS = _INFO.num_subcores
NW = NC * NS
PER_W = N // NW
CHUNK = 64
NCHUNK = PER_W // CHUNK
LANES = 16
VPR = EMBED_DIM // LANES
NBUF = 4
SKEW = 3

TBLOCK = 512
TGRID = 977
HALF_V = TGRID * TBLOCK

_mesh = plsc.VectorSubcoreMesh(core_axis_name="c", subcore_axis_name="s")


def _xpose_body(lo_ref, hi_ref, out_ref):
    out_ref[:, :EMBED_DIM] = jnp.transpose(lo_ref[...])
    out_ref[:, EMBED_DIM:] = jnp.transpose(hi_ref[...])


_xpose = pl.pallas_call(
    _xpose_body,
    grid=(TGRID,),
    in_specs=[
        pl.BlockSpec((EMBED_DIM, TBLOCK), lambda i: (0, i)),
        pl.BlockSpec((EMBED_DIM, TBLOCK), lambda i: (0, i + TGRID)),
    ],
    out_specs=pl.BlockSpec((TBLOCK, WIDE), lambda i: (i, 0)),
    out_shape=jax.ShapeDtypeStruct((HALF_V, WIDE), jnp.float32),
)


@functools.partial(
    pl.kernel,
    out_type=jax.ShapeDtypeStruct((N // 2, WIDE), jnp.float32),
    mesh=_mesh,
    compiler_params=pltpu.CompilerParams(use_tc_tiling_on_sc=True),
    scratch_types=[
        pltpu.VMEM((PER_W,), jnp.int32),
        pltpu.VMEM((PER_W,), jnp.int32),
        pltpu.VMEM((MAX_LEN, WIDE), jnp.float32),
        [pltpu.VMEM((CHUNK, WIDE), jnp.float32) for _ in range(NBUF)],
        [pltpu.VMEM((CHUNK // 2, WIDE), jnp.float32) for _ in range(NBUF)],
        [pltpu.SemaphoreType.DMA for _ in range(NBUF)],
        [pltpu.SemaphoreType.DMA for _ in range(NBUF)],
    ],
)
def _embed_sc(widx_hbm, off_hbm, tokw_hbm, posw_hbm, out_hbm,
              widx_v, off_v, pos_v, wide, outb, gsem, ssem):
    wid = lax.axis_index("s") * NC + lax.axis_index("c")
    base = wid * PER_W

    pltpu.sync_copy(widx_hbm.at[pl.ds(pl.multiple_of(base, 8), PER_W)], widx_v)
    pltpu.sync_copy(off_hbm.at[pl.ds(pl.multiple_of(base, 8), PER_W)], off_v)
    pltpu.sync_copy(posw_hbm, pos_v.at[pl.ds(0, MAX_LEN // 2)])
    pltpu.sync_copy(posw_hbm, pos_v.at[pl.ds(MAX_LEN // 2, MAX_LEN // 2)])

    def gather_start(c, b):
        pltpu.make_async_copy(
            tokw_hbm.at[widx_v.at[pl.ds(pl.multiple_of(c * CHUNK, 8), CHUNK)]],
            wide[b], gsem[b],
        ).start()

    def gather_wait(b):
        pltpu.make_async_copy(
            tokw_hbm.at[widx_v.at[pl.ds(0, CHUNK)]], wide[b], gsem[b]
        ).wait()

    def scatter_start(c, b):
        pltpu.make_async_copy(
            outb[b],
            out_hbm.at[pl.ds(pl.multiple_of((base + c * CHUNK) // 2, 8), CHUNK // 2)],
            ssem[b],
        ).start()

    def scatter_wait(b):
        pltpu.make_async_copy(
            outb[b], out_hbm.at[pl.ds(pl.multiple_of(base // 2, 8), CHUNK // 2)], ssem[b]
        ).wait()

    for b in range(SKEW):
        gather_start(b, b)

    def outer(g, carry):
        for b in range(NBUF):
            c = NBUF * g + b
            gather_wait(b)
            coff = c * CHUNK
            pw = lax.rem(coff, MAX_LEN) // 2

            def group_body(gi, carry2):
                ov = off_v[pl.ds(coff + gi * LANES, LANES)]
                for m in range(8):
                    u = gi * 8 + m
                    t0 = gi * LANES + 2 * m
                    t1 = t0 + 1
                    o0 = ov[2 * m]
                    o1 = ov[2 * m + 1]
                    pr = pw + u
                    for j in range(VPR):
                        lo = j * LANES
                        hi = EMBED_DIM + lo
                        outb[b][u, pl.ds(lo, LANES)] = (
                            wide[b][t0, pl.ds(o0 + lo, LANES)]
                            + pos_v[pr, pl.ds(lo, LANES)]
                        )
                        outb[b][u, pl.ds(hi, LANES)] = (
                            wide[b][t1, pl.ds(o1 + lo, LANES)]
                            + pos_v[pr, pl.ds(hi, LANES)]
                        )
                return carry2

            lax.fori_loop(0, CHUNK // LANES, group_body, 0)
            scatter_start(c, b)

            f = c + SKEW
            bf = (b + SKEW) % NBUF

            @pl.when(jnp.logical_and(c >= 1, f < NCHUNK))
            def _():
                scatter_wait(bf)

            @pl.when(f < NCHUNK)
            def _():
                gather_start(f, bf)

        return carry

    lax.fori_loop(0, NCHUNK // NBUF, outer, 0)

    for b in range(NBUF):
        scatter_wait(b)


def kernel(inputs, token_table, position_table):
    flat = inputs.reshape(N)
    widx = jnp.where(flat < HALF_V, flat, flat - HALF_V)
    off = jnp.where(flat < HALF_V, 0, EMBED_DIM).astype(jnp.int32)
    tokt = token_table.T
    tokw = _xpose(tokt, tokt)
    posw = position_table.reshape(MAX_LEN // 2, WIDE)
    out = _embed_sc(widx, off, tokw, posw)
    return out.reshape(BATCH, SEQ_LEN, EMBED_DIM)


# speedup vs baseline: 1.1730x; 1.1730x over previous
import functools

import jax
import jax.numpy as jnp
from jax import lax
from jax.experimental import pallas as pl
from jax.experimental.pallas import tpu as pltpu
from jax.experimental.pallas import tpu_sc as plsc

VOCAB = 1000000
MAX_LEN = 200
EMBED_DIM = 64
BATCH = 1024
SEQ_LEN = 200

N = BATCH * SEQ_LEN
_INFO = plsc.get_sparse_core_info()
NC = _INFO.num_cores
NS = _INFO.num_subcores
NW = NC * NS
SEQ_PER_W = BATCH // NW
PER_W = N // NW
CHUNK = SEQ_LEN
SPLIT = 104
LANES = 16
VPR = EMBED_DIM // LANES
NBUF = 4
SKEW = 3

_mesh = plsc.VectorSubcoreMesh(core_axis_name="c", subcore_axis_name="s")


@functools.partial(
    pl.kernel,
    out_type=jax.ShapeDtypeStruct((BATCH, SEQ_LEN, EMBED_DIM), jnp.float32),
    mesh=_mesh,
    compiler_params=pltpu.CompilerParams(use_tc_tiling_on_sc=False),
    scratch_types=[
        pltpu.VMEM((PER_W,), jnp.int32),
        pltpu.VMEM((MAX_LEN, EMBED_DIM), jnp.float32),
        [pltpu.VMEM((CHUNK, EMBED_DIM), jnp.float32) for _ in range(NBUF)],
        [pltpu.SemaphoreType.DMA for _ in range(NBUF)],
        [pltpu.SemaphoreType.DMA for _ in range(NBUF)],
    ],
)
def _embed_sc(idx_hbm, tok_hbm, pos_hbm, out_hbm, idx_v, pos_v, data, gsem, ssem):
    wid = lax.axis_index("s") * NC + lax.axis_index("c")
    base = wid * PER_W
    seq0 = wid * SEQ_PER_W

    pltpu.sync_copy(idx_hbm.at[pl.ds(pl.multiple_of(base, 8), PER_W)], idx_v)
    pltpu.sync_copy(pos_hbm, pos_v)

    def gather_start(c, b):
        lo = pl.multiple_of(c * CHUNK, 8)
        hi = pl.multiple_of(c * CHUNK + SPLIT, 8)
        pltpu.make_async_copy(
            tok_hbm.at[idx_v.at[pl.ds(lo, SPLIT)]],
            data[b].at[pl.ds(0, SPLIT)], gsem[b],
        ).start()
        pltpu.make_async_copy(
            tok_hbm.at[idx_v.at[pl.ds(hi, CHUNK - SPLIT)]],
            data[b].at[pl.ds(SPLIT, CHUNK - SPLIT)], gsem[b],
        ).start()

    def gather_wait(b):
        pltpu.make_async_copy(
            tok_hbm.at[idx_v.at[pl.ds(0, CHUNK)]], data[b], gsem[b]
        ).wait()

    def scatter_start(c, b):
        pltpu.make_async_copy(data[b], out_hbm.at[seq0 + c], ssem[b]).start()

    def scatter_wait(b):
        pltpu.make_async_copy(data[b], out_hbm.at[seq0], ssem[b]).wait()

    for b in range(SKEW):
        gather_start(b, b)

    def outer(g, carry):
        for b in range(NBUF):
            c = NBUF * g + b
            gather_wait(b)

            def row_body(r, carry2):
                for j in range(VPR):
                    sl = pl.ds(j * LANES, LANES)
                    data[b][r, sl] = data[b][r, sl] + pos_v[r, sl]
                return carry2

            lax.fori_loop(0, CHUNK, row_body, 0, unroll=4)
            scatter_start(c, b)

            f = c + SKEW
            bf = (b + SKEW) % NBUF

            @pl.when(jnp.logical_and(c >= 1, f < SEQ_PER_W))
            def _():
                scatter_wait(bf)

            @pl.when(f < SEQ_PER_W)
            def _():
                gather_start(f, bf)

        return carry

    lax.fori_loop(0, SEQ_PER_W // NBUF, outer, 0)

    for b in range(NBUF):
        scatter_wait(b)


def kernel(inputs, token_table, position_table):
    flat = inputs.reshape(N)
    return _embed_sc(flat, token_table, position_table)
